# baseline (speedup 1.0000x reference)
import functools

import jax
import jax.numpy as jnp
from jax import lax
from jax.experimental import pallas as pl
from jax.experimental.pallas import tpu as pltpu
from jax.experimental.pallas import tpu_sc as plsc

_INFO = plsc.get_sparse_core_info()
_NC, _NS = _INFO.num_cores, _INFO.num_subcores
_NW = _NC * _NS

_CHUNK = 1600
_PAD_D = 128
_TCOLS = 4
_TPW = 62


@functools.lru_cache(maxsize=None)
def _make_transpose(V, D):
    n_tj = D // 8
    n_ti = (V + 127) // 128
    lanes = _TCOLS * 128
    bwords = lanes * D
    last_start = n_ti - _TCOLS
    mesh = plsc.VectorSubcoreMesh(core_axis_name="c", subcore_axis_name="s")

    @functools.partial(
        pl.kernel,
        mesh=mesh,
        out_type=jax.ShapeDtypeStruct((n_ti * 128 * D,), jnp.float32),
        scratch_types=[
            pltpu.VMEM((D, lanes), jnp.float32),
            pltpu.VMEM((D, lanes), jnp.float32),
            pltpu.VMEM((bwords,), jnp.float32),
            pltpu.VMEM((bwords,), jnp.float32),
            pltpu.SemaphoreType.DMA,
            pltpu.SemaphoreType.DMA,
            pltpu.SemaphoreType.DMA,
            pltpu.SemaphoreType.DMA,
        ],
        compiler_params=pltpu.CompilerParams(needs_layout_passes=False),
    )
    def transpose_kernel(tab_t, rows1d, tiles0, tiles1, blk0, blk1,
                         gsem0, gsem1, ssem0, ssem1):
        wid = lax.axis_index("s") * _NC + lax.axis_index("c")
        tiles = (tiles0, tiles1)
        blk = (blk0, blk1)
        gsem = (gsem0, gsem1)
        ssem = (ssem0, ssem1)

        i16d = lax.iota(jnp.int32, 16) * D

        def start_col(g):
            return jnp.minimum((wid * _TPW + g) * _TCOLS, last_start)

        def start_in(g, b):
            lane0 = start_col(g) * 128
            for tj in range(n_tj):
                pltpu.async_copy(
                    tab_t.at[pl.ds(8 * tj, 8), pl.ds(lane0, lanes)],
                    tiles[b].at[pl.ds(8 * tj, 8), :], gsem[b])

        def wait_in(b):
            for tj in range(n_tj):
                pltpu.make_async_copy(
                    tab_t.at[pl.ds(0, 8), pl.ds(0, lanes)],
                    tiles[b].at[pl.ds(0, 8), :], gsem[b]).wait()

        def start_out(g, b):
            off = start_col(g) * 128 * D
            pltpu.async_copy(blk[b], rows1d.at[pl.ds(off, bwords)],
                             ssem[b])

        def wait_out(b):
            pltpu.make_async_copy(rows1d.at[pl.ds(0, bwords)], blk[b],
                                  ssem[b]).wait()

        start_in(0, 0)

        def group(gg, carry):
            for b in (0, 1):
                g = 2 * gg + b
                wait_in(b)
                nb = 1 - b

                @pl.when(g + 1 < _TPW)
                def _():
                    start_in(g + 1, nb)

                @pl.when(g >= 2)
                def _():
                    wait_out(b)

                @plsc.parallel_loop(0, lanes // 16, unroll=4)
                def _(c):
                    l0 = c * 16
                    ibase = i16d + l0 * D
                    for j in range(D):
                        v = tiles[b][j, pl.ds(l0, 16)]
                        plsc.store_scatter(blk[b], [ibase + j], v)

                start_out(g, b)
            return carry

        lax.fori_loop(0, _TPW // 2, group, 0)
        wait_out(0)
        wait_out(1)

    return transpose_kernel


@functools.lru_cache(maxsize=None)
def _make_gather(V, D, B):
    assert B % _NW == 0
    b_per_w = B // _NW
    assert b_per_w % (2 * _CHUNK) == 0
    n_steps = b_per_w // _CHUNK
    mesh = plsc.VectorSubcoreMesh(core_axis_name="c", subcore_axis_name="s")

    @functools.partial(
        pl.kernel,
        mesh=mesh,
        out_type=jax.ShapeDtypeStruct((B, _PAD_D), jnp.float32),
        scratch_types=[
            pltpu.VMEM((b_per_w,), jnp.int32),
            pltpu.VMEM((_CHUNK, D), jnp.float32),
            pltpu.VMEM((_CHUNK, D), jnp.float32),
            pltpu.SemaphoreType.DMA,
            pltpu.SemaphoreType.DMA,
            pltpu.SemaphoreType.DMA,
            pltpu.SemaphoreType.DMA,
        ],
        compiler_params=pltpu.CompilerParams(use_tc_tiling_on_sc=False),
    )
    def gather_kernel(table_hbm, idx_hbm, out_hbm, idx_v, rows0, rows1,
                      gsem0, gsem1, ssem0, ssem1):
        wid = lax.axis_index("s") * _NC + lax.axis_index("c")
        base = wid * b_per_w
        rows = (rows0, rows1)
        gsem = (gsem0, gsem1)
        ssem = (ssem0, ssem1)

        pltpu.sync_copy(idx_hbm.at[pl.ds(base, b_per_w)], idx_v)

        def start_gather(i, b):
            pltpu.async_copy(
                table_hbm.at[idx_v.at[pl.ds(i * _CHUNK, _CHUNK)]],
                rows[b], gsem[b])

        def wait_gather(b):
            pltpu.make_async_copy(
                table_hbm.at[pl.ds(0, _CHUNK)], rows[b], gsem[b]).wait()

        def start_store(i, b):
            off = base + i * _CHUNK
            pltpu.async_copy(
                rows[b], out_hbm.at[pl.ds(off, _CHUNK), pl.ds(0, D)],
                ssem[b])

        def wait_store(b):
            pltpu.make_async_copy(
                out_hbm.at[pl.ds(0, _CHUNK), pl.ds(0, D)], rows[b],
                ssem[b]).wait()

        start_gather(0, 0)

        def group(g, carry):
            for b in (0, 1):
                i = 2 * g + b
                wait_gather(b)
                nb = 1 - b

                @pl.when(jnp.logical_and(i >= 1, i + 1 < n_steps))
                def _():
                    wait_store(nb)

                @pl.when(i + 1 < n_steps)
                def _():
                    start_gather(i + 1, nb)

                start_store(i, b)
            return carry

        lax.fori_loop(0, n_steps // 2, group, 0)
        wait_store(0)
        wait_store(1)

    return gather_kernel


def kernel(x, table):
    B = x.shape[0] * x.shape[1]
    V, D = table.shape
    flat = jnp.reshape(x, (B,)).astype(jnp.int32)
    rows1d = _make_transpose(V, D)(jnp.transpose(table))
    v_pad = rows1d.shape[0] // D
    rows_rm = jnp.reshape(rows1d, (v_pad, D))
    out_pad = _make_gather(v_pad, D, B)(rows_rm, flat)
    return out_pad[:, :D]


# speedup vs baseline: 1.0594x; 1.0594x over previous
import functools

import jax
import jax.numpy as jnp
from jax import lax
from jax.experimental import pallas as pl
from jax.experimental.pallas import tpu as pltpu
from jax.experimental.pallas import tpu_sc as plsc

_INFO = plsc.get_sparse_core_info()
_NC, _NS = _INFO.num_cores, _INFO.num_subcores
_NW = _NC * _NS

_CHUNK = 1600
_PAD_D = 128
_TCOLS = 4
_TPW = 62


@functools.lru_cache(maxsize=None)
def _make_transpose(V, D):
    n_tj = D // 8
    n_ti = (V + 127) // 128
    lanes = _TCOLS * 128
    bwords = lanes * D
    last_start = n_ti - _TCOLS
    mesh = plsc.VectorSubcoreMesh(core_axis_name="c", subcore_axis_name="s")

    @functools.partial(
        pl.kernel,
        mesh=mesh,
        out_type=jax.ShapeDtypeStruct((n_ti * 128 * D,), jnp.float32),
        scratch_types=[
            pltpu.VMEM((D, lanes), jnp.float32),
            pltpu.VMEM((D, lanes), jnp.float32),
            pltpu.VMEM((bwords,), jnp.float32),
            pltpu.VMEM((bwords,), jnp.float32),
            pltpu.SemaphoreType.DMA,
            pltpu.SemaphoreType.DMA,
            pltpu.SemaphoreType.DMA,
            pltpu.SemaphoreType.DMA,
        ],
        compiler_params=pltpu.CompilerParams(needs_layout_passes=False),
    )
    def transpose_kernel(tab_t, rows1d, tiles0, tiles1, blk0, blk1,
                         gsem0, gsem1, ssem0, ssem1):
        wid = lax.axis_index("s") * _NC + lax.axis_index("c")
        tiles = (tiles0, tiles1)
        blk = (blk0, blk1)
        gsem = (gsem0, gsem1)
        ssem = (ssem0, ssem1)

        i16d = lax.iota(jnp.int32, 16) * D

        def start_col(g):
            return jnp.minimum((wid * _TPW + g) * _TCOLS, last_start)

        def start_in(g, b):
            lane0 = start_col(g) * 128
            for tj in range(n_tj):
                pltpu.async_copy(
                    tab_t.at[pl.ds(8 * tj, 8), pl.ds(lane0, lanes)],
                    tiles[b].at[pl.ds(8 * tj, 8), :], gsem[b])

        def wait_in(b):
            for tj in range(n_tj):
                pltpu.make_async_copy(
                    tab_t.at[pl.ds(0, 8), pl.ds(0, lanes)],
                    tiles[b].at[pl.ds(0, 8), :], gsem[b]).wait()

        def start_out(g, b):
            off = start_col(g) * 128 * D
            pltpu.async_copy(blk[b], rows1d.at[pl.ds(off, bwords)],
                             ssem[b])

        def wait_out(b):
            pltpu.make_async_copy(rows1d.at[pl.ds(0, bwords)], blk[b],
                                  ssem[b]).wait()

        start_in(0, 0)

        def group(gg, carry):
            for b in (0, 1):
                g = 2 * gg + b
                wait_in(b)
                nb = 1 - b

                @pl.when(g + 1 < _TPW)
                def _():
                    start_in(g + 1, nb)

                @pl.when(g >= 2)
                def _():
                    wait_out(b)

                @plsc.parallel_loop(0, lanes // 16, unroll=2)
                def _(c):
                    l0 = c * 16
                    ibase = i16d + l0 * D
                    for j in range(D):
                        v = tiles[b][j, pl.ds(l0, 16)]
                        plsc.store_scatter(blk[b], [ibase + j], v)

                start_out(g, b)
            return carry

        lax.fori_loop(0, _TPW // 2, group, 0)
        wait_out(0)
        wait_out(1)

    return transpose_kernel


@functools.lru_cache(maxsize=None)
def _make_gather(V, D, B):
    assert B % _NW == 0
    b_per_w = B // _NW
    assert b_per_w % (2 * _CHUNK) == 0
    n_steps = b_per_w // _CHUNK
    mesh = plsc.VectorSubcoreMesh(core_axis_name="c", subcore_axis_name="s")

    @functools.partial(
        pl.kernel,
        mesh=mesh,
        out_type=jax.ShapeDtypeStruct((B, _PAD_D), jnp.float32),
        scratch_types=[
            pltpu.VMEM((b_per_w,), jnp.int32),
            pltpu.VMEM((_CHUNK, D), jnp.float32),
            pltpu.VMEM((_CHUNK, D), jnp.float32),
            pltpu.SemaphoreType.DMA,
            pltpu.SemaphoreType.DMA,
            pltpu.SemaphoreType.DMA,
            pltpu.SemaphoreType.DMA,
        ],
        compiler_params=pltpu.CompilerParams(use_tc_tiling_on_sc=False),
    )
    def gather_kernel(table_hbm, idx_hbm, out_hbm, idx_v, rows0, rows1,
                      gsem0, gsem1, ssem0, ssem1):
        wid = lax.axis_index("s") * _NC + lax.axis_index("c")
        base = wid * b_per_w
        rows = (rows0, rows1)
        gsem = (gsem0, gsem1)
        ssem = (ssem0, ssem1)

        pltpu.sync_copy(idx_hbm.at[pl.ds(base, b_per_w)], idx_v)

        def start_gather(i, b):
            pltpu.async_copy(
                table_hbm.at[idx_v.at[pl.ds(i * _CHUNK, _CHUNK)]],
                rows[b], gsem[b])

        def wait_gather(b):
            pltpu.make_async_copy(
                table_hbm.at[pl.ds(0, _CHUNK)], rows[b], gsem[b]).wait()

        def start_store(i, b):
            off = base + i * _CHUNK
            pltpu.async_copy(
                rows[b], out_hbm.at[pl.ds(off, _CHUNK), pl.ds(0, D)],
                ssem[b])

        def wait_store(b):
            pltpu.make_async_copy(
                out_hbm.at[pl.ds(0, _CHUNK), pl.ds(0, D)], rows[b],
                ssem[b]).wait()

        start_gather(0, 0)

        def group(g, carry):
            for b in (0, 1):
                i = 2 * g + b
                wait_gather(b)
                nb = 1 - b

                @pl.when(jnp.logical_and(i >= 1, i + 1 < n_steps))
                def _():
                    wait_store(nb)

                @pl.when(i + 1 < n_steps)
                def _():
                    start_gather(i + 1, nb)

                start_store(i, b)
            return carry

        lax.fori_loop(0, n_steps // 2, group, 0)
        wait_store(0)
        wait_store(1)

    return gather_kernel


def kernel(x, table):
    B = x.shape[0] * x.shape[1]
    V, D = table.shape
    flat = jnp.reshape(x, (B,)).astype(jnp.int32)
    rows1d = _make_transpose(V, D)(jnp.transpose(table))
    v_pad = rows1d.shape[0] // D
    rows_rm = jnp.reshape(rows1d, (v_pad, D))
    out_pad = _make_gather(v_pad, D, B)(rows_rm, flat)
    return out_pad[:, :D]
